# baseline (speedup 1.0000x reference)
import functools

import jax
import jax.numpy as jnp
from jax import lax
from jax.experimental import pallas as pl
from jax.experimental.pallas import tpu as pltpu


def _pos_kernel(row_ref, col_ref, out_ref, plane_ref, sems, *, b, h, w, d):
    colT = col_ref[:w, :].T
    rowT = row_ref[:h, :].T
    plane_ref[:d, :, :] = jnp.broadcast_to(colT[:, None, :], (d, h, w))
    plane_ref[d:, :, :] = jnp.broadcast_to(rowT[:, :, None], (d, h, w))

    copies = [
        pltpu.make_async_copy(plane_ref, out_ref.at[i], sems.at[i])
        for i in range(b)
    ]
    for c in copies:
        c.start()
    for c in copies:
        c.wait()


def kernel(x, row_weight, col_weight):
    b = x.shape[0]
    h, w = x.shape[-2], x.shape[-1]
    d = row_weight.shape[1]

    body = functools.partial(_pos_kernel, b=b, h=h, w=w, d=d)
    out = pl.pallas_call(
        body,
        in_specs=[
            pl.BlockSpec(memory_space=pltpu.MemorySpace.VMEM),
            pl.BlockSpec(memory_space=pltpu.MemorySpace.VMEM),
        ],
        out_specs=pl.BlockSpec(memory_space=pltpu.MemorySpace.HBM),
        out_shape=jax.ShapeDtypeStruct((b, 2 * d, h, w), jnp.float32),
        scratch_shapes=[
            pltpu.VMEM((2 * d, h, w), jnp.float32),
            pltpu.SemaphoreType.DMA((b,)),
        ],
    )(row_weight, col_weight)
    return out
